# baseline (speedup 1.0000x reference)
import functools

import jax
import jax.numpy as jnp
from jax import lax
from jax.experimental import pallas as pl
from jax.experimental.pallas import tpu as pltpu
from jax.experimental.pallas import tpu_sc as plsc

NUM_CORES = 2
NUM_SUBCORES = 16
NUM_WORKERS = NUM_CORES * NUM_SUBCORES
CHUNK = 128
EMB = 128
LANES = 16


def _sc_gather_add(attr_table, obj_table, va, vo, p_pad):
    rows_per_w = p_pad // NUM_WORKERS
    n_chunks = rows_per_w // CHUNK
    mesh = plsc.VectorSubcoreMesh(core_axis_name="c", subcore_axis_name="s")

    @functools.partial(
        pl.kernel,
        mesh=mesh,
        out_type=jax.ShapeDtypeStruct((p_pad, EMB), jnp.float32),
        scratch_types=[
            pltpu.VMEM((CHUNK,), jnp.int32),
            pltpu.VMEM((CHUNK,), jnp.int32),
            pltpu.VMEM((CHUNK, EMB), jnp.float32),
            pltpu.VMEM((CHUNK, EMB), jnp.float32),
            pltpu.SemaphoreType.DMA,
            pltpu.SemaphoreType.DMA,
        ],
    )
    def body(attr_hbm, obj_hbm, va_hbm, vo_hbm, out_hbm,
             ia_v, io_v, ra_v, rb_v, sem_a, sem_b):
        wid = lax.axis_index("s") * NUM_CORES + lax.axis_index("c")
        base = wid * rows_per_w

        def chunk_step(i, carry):
            off = base + i * CHUNK
            pltpu.sync_copy(va_hbm.at[pl.ds(off, CHUNK)], ia_v)
            pltpu.sync_copy(vo_hbm.at[pl.ds(off, CHUNK)], io_v)
            cpa = pltpu.async_copy(attr_hbm.at[ia_v], ra_v, sem_a)
            cpb = pltpu.async_copy(obj_hbm.at[io_v], rb_v, sem_b)
            cpa.wait()
            cpb.wait()

            def row_step(r, c2):
                for j in range(EMB // LANES):
                    sl = pl.ds(j * LANES, LANES)
                    ra_v[r, sl] = ra_v[r, sl] + rb_v[r, sl]
                return c2

            lax.fori_loop(0, CHUNK, row_step, 0)
            pltpu.sync_copy(ra_v, out_hbm.at[pl.ds(off, CHUNK)])
            return carry

        lax.fori_loop(0, n_chunks, chunk_step, 0)

    return body(attr_table, obj_table, va, vo)


def _tc_scores(x, pair, n_pairs, tile_p):
    batch = x.shape[0]
    grid = (n_pairs + tile_p - 1) // tile_p

    def body(x_ref, p_ref, o_ref):
        xv = x_ref[...]
        xn = xv * (1.0 / (jnp.sqrt(jnp.sum(xv * xv, axis=1, keepdims=True)) + 1e-8))
        pv = p_ref[...]
        pinv = 1.0 / (jnp.sqrt(jnp.sum(pv * pv, axis=1, keepdims=True)) + 1e-8)
        pn = pv * pinv
        o_ref[...] = lax.dot_general(
            xn, pn, (((1,), (1,)), ((), ())),
            preferred_element_type=jnp.float32)

    return pl.pallas_call(
        body,
        grid=(grid,),
        in_specs=[
            pl.BlockSpec((batch, EMB), lambda j: (0, 0)),
            pl.BlockSpec((tile_p, EMB), lambda j: (j, 0)),
        ],
        out_specs=pl.BlockSpec((batch, tile_p), lambda j: (0, j)),
        out_shape=jax.ShapeDtypeStruct((batch, n_pairs), jnp.float32),
    )(x, pair)


def kernel(x, val_attrs, val_objs, attr_table, obj_table):
    n_pairs = val_attrs.shape[0]
    quantum = NUM_WORKERS * CHUNK
    p_pad = ((n_pairs + quantum - 1) // quantum) * quantum
    va = jnp.pad(val_attrs.astype(jnp.int32), (0, p_pad - n_pairs))
    vo = jnp.pad(val_objs.astype(jnp.int32), (0, p_pad - n_pairs))
    pair = _sc_gather_add(attr_table, obj_table, va, vo, p_pad)
    return _tc_scores(x, pair, n_pairs, tile_p=512)
